# initial kernel scaffold (unmeasured)
import jax
import jax.numpy as jnp
from jax import lax
from jax.experimental import pallas as pl
from jax.experimental.pallas import tpu as pltpu


def kernel(
    x,
):
    def body(*refs):
        pass

    out_shape = jax.ShapeDtypeStruct(..., jnp.float32)
    return pl.pallas_call(body, out_shape=out_shape)(...)


# baseline (device time: 80041 ns/iter reference)
import jax
import jax.numpy as jnp
from jax import lax
from jax.experimental import pallas as pl
from jax.experimental.pallas import tpu as pltpu

N_DEV = 4


def kernel(x):
    m, n = x.shape

    def body(x_ref, out_ref, comm_ref, send_sems, recv_sems):
        my_pos = lax.axis_index("i")
        left = lax.rem(my_pos - 1 + N_DEV, N_DEV)
        right = lax.rem(my_pos + 1, N_DEV)

        barrier_sem = pltpu.get_barrier_semaphore()
        for nbr in [left, right]:
            pl.semaphore_signal(
                barrier_sem, inc=1,
                device_id=(nbr,), device_id_type=pl.DeviceIdType.MESH,
            )
        pl.semaphore_wait(barrier_sem, 2)

        out_ref[:, :] = x_ref[:, :]
        comm_ref[0, :, :] = x_ref[:, :].astype(jnp.bfloat16)

        for h in range(N_DEV - 1):
            send_slot = h % 2
            recv_slot = (h + 1) % 2
            rdma = pltpu.make_async_remote_copy(
                src_ref=comm_ref.at[send_slot],
                dst_ref=comm_ref.at[recv_slot],
                send_sem=send_sems.at[send_slot],
                recv_sem=recv_sems.at[recv_slot],
                device_id=(right,),
                device_id_type=pl.DeviceIdType.MESH,
            )
            rdma.start()
            rdma.wait()
            out_ref[:, :] += comm_ref[recv_slot, :, :].astype(jnp.float32)

    return pl.pallas_call(
        body,
        out_shape=jax.ShapeDtypeStruct((m, n), jnp.float32),
        in_specs=[pl.BlockSpec(memory_space=pltpu.VMEM)],
        out_specs=pl.BlockSpec(memory_space=pltpu.VMEM),
        scratch_shapes=[
            pltpu.VMEM((2, m, n), jnp.bfloat16),
            pltpu.SemaphoreType.DMA((2,)),
            pltpu.SemaphoreType.DMA((2,)),
        ],
        compiler_params=pltpu.CompilerParams(collective_id=0),
    )(x)


# device time: 33379 ns/iter; 2.3979x vs baseline; 2.3979x over previous
import jax
import jax.numpy as jnp
from jax import lax
from jax.experimental import pallas as pl
from jax.experimental.pallas import tpu as pltpu

N_DEV = 4


def kernel(x):
    m, n = x.shape
    q = m // N_DEV

    def body(
        x_ref,
        out_ref,
        xb_ref,
        rs_recv_ref,
        ag_send_ref,
        rs_send_sems,
        rs_recv_sems,
        ag_send_sems,
        ag_recv_sems,
    ):
        my = lax.axis_index("i")

        barrier_sem = pltpu.get_barrier_semaphore()
        for j in range(1, N_DEV):
            pl.semaphore_signal(
                barrier_sem, inc=1,
                device_id=(lax.rem(my + j, N_DEV),),
                device_id_type=pl.DeviceIdType.MESH,
            )
        pl.semaphore_wait(barrier_sem, N_DEV - 1)

        xb_ref[:, :] = x_ref[:, :].astype(jnp.bfloat16)

        rs_sends = []
        for j in range(N_DEV - 1):
            t = lax.rem(my + 1 + j, N_DEV)
            slot = N_DEV - 2 - j
            rdma = pltpu.make_async_remote_copy(
                src_ref=xb_ref.at[pl.ds(t * q, q), :],
                dst_ref=rs_recv_ref.at[slot],
                send_sem=rs_send_sems.at[j],
                recv_sem=rs_recv_sems.at[slot],
                device_id=(t,),
                device_id_type=pl.DeviceIdType.MESH,
            )
            rdma.start()
            rs_sends.append(rdma)

        for slot in range(N_DEV - 1):
            pltpu.make_async_remote_copy(
                src_ref=rs_recv_ref.at[slot],
                dst_ref=rs_recv_ref.at[slot],
                send_sem=rs_send_sems.at[slot],
                recv_sem=rs_recv_sems.at[slot],
                device_id=(my,),
                device_id_type=pl.DeviceIdType.MESH,
            ).wait_recv()

        acc = x_ref[pl.ds(my * q, q), :]
        for slot in range(N_DEV - 1):
            acc = acc + rs_recv_ref[slot, :, :].astype(jnp.float32)
        red = acc.astype(jnp.bfloat16)
        out_ref[pl.ds(my * q, q), :] = red
        ag_send_ref[:, :] = red

        ag_sends = []
        for j in range(N_DEV - 1):
            t = lax.rem(my + 1 + j, N_DEV)
            slot = N_DEV - 2 - j
            rdma = pltpu.make_async_remote_copy(
                src_ref=ag_send_ref,
                dst_ref=out_ref.at[pl.ds(my * q, q), :],
                send_sem=ag_send_sems.at[j],
                recv_sem=ag_recv_sems.at[slot],
                device_id=(t,),
                device_id_type=pl.DeviceIdType.MESH,
            )
            rdma.start()
            ag_sends.append(rdma)

        for slot in range(N_DEV - 1):
            s = lax.rem(my + 1 + slot, N_DEV)
            pltpu.make_async_remote_copy(
                src_ref=ag_send_ref,
                dst_ref=out_ref.at[pl.ds(s * q, q), :],
                send_sem=ag_send_sems.at[0],
                recv_sem=ag_recv_sems.at[slot],
                device_id=(my,),
                device_id_type=pl.DeviceIdType.MESH,
            ).wait_recv()

        for r in rs_sends + ag_sends:
            r.wait_send()

    return pl.pallas_call(
        body,
        out_shape=jax.ShapeDtypeStruct((m, n), jnp.bfloat16),
        in_specs=[pl.BlockSpec(memory_space=pltpu.VMEM)],
        out_specs=pl.BlockSpec(memory_space=pltpu.VMEM),
        scratch_shapes=[
            pltpu.VMEM((m, n), jnp.bfloat16),
            pltpu.VMEM((N_DEV - 1, q, n), jnp.bfloat16),
            pltpu.VMEM((q, n), jnp.bfloat16),
            pltpu.SemaphoreType.DMA((N_DEV - 1,)),
            pltpu.SemaphoreType.DMA((N_DEV - 1,)),
            pltpu.SemaphoreType.DMA((N_DEV - 1,)),
            pltpu.SemaphoreType.DMA((N_DEV - 1,)),
        ],
        compiler_params=pltpu.CompilerParams(collective_id=0),
    )(x)


# device time: 30129 ns/iter; 2.6566x vs baseline; 1.1079x over previous
import jax
import jax.numpy as jnp
from jax import lax
from jax.experimental import pallas as pl
from jax.experimental.pallas import tpu as pltpu

N_DEV = 4


def kernel(x):
    m, n = x.shape
    hm, hn = m // 2, n // 2
    qm = m // 4

    def body(
        x_ref,
        out_ref,
        xb_ref,
        st1a,
        st1b,
        st2a,
        st2b,
        red1a,
        red1b,
        send_sems,
        recv_sems,
    ):
        my = lax.axis_index("i")
        b0 = my & 1
        b1 = my >> 1
        pa1 = my ^ 1
        pa2 = my ^ 3

        barrier_sem = pltpu.get_barrier_semaphore()
        for nbr in [pa1, pa2]:
            pl.semaphore_signal(
                barrier_sem, inc=1,
                device_id=(nbr,), device_id_type=pl.DeviceIdType.MESH,
            )
        pl.semaphore_wait(barrier_sem, 2)

        xb_ref[:, :] = x_ref[:, :].astype(jnp.bfloat16)

        sends = []

        def exch(idx, src, dst, tgt):
            r = pltpu.make_async_remote_copy(
                src_ref=src,
                dst_ref=dst,
                send_sem=send_sems.at[idx],
                recv_sem=recv_sems.at[idx],
                device_id=(tgt,),
                device_id_type=pl.DeviceIdType.MESH,
            )
            r.start()
            sends.append(r)
            return r

        keep_lo_a = b0 == b1
        off_keep_a = jnp.where(keep_lo_a, 0, hm)
        off_send_a = jnp.where(keep_lo_a, hm, 0)
        keep_lo_b = b1 == 0
        off_keep_b = jnp.where(keep_lo_b, 0, hm)
        off_send_b = jnp.where(keep_lo_b, hm, 0)

        r1a = exch(0, xb_ref.at[pl.ds(off_send_a, hm), pl.ds(0, hn)], st1a, pa1)
        r1b = exch(1, xb_ref.at[pl.ds(off_send_b, hm), pl.ds(hn, hn)], st1b, pa2)
        r1a.wait_recv()
        r1b.wait_recv()
        red1a[:, :] = (
            xb_ref[pl.ds(off_keep_a, hm), pl.ds(0, hn)].astype(jnp.float32)
            + st1a[:, :].astype(jnp.float32)
        ).astype(jnp.bfloat16)
        red1b[:, :] = (
            xb_ref[pl.ds(off_keep_b, hm), pl.ds(hn, hn)].astype(jnp.float32)
            + st1b[:, :].astype(jnp.float32)
        ).astype(jnp.bfloat16)

        keep_first_a = b1 == 0
        k2a = jnp.where(keep_first_a, 0, qm)
        s2a = jnp.where(keep_first_a, qm, 0)
        keep_first_b = b0 == 0
        k2b = jnp.where(keep_first_b, 0, qm)
        s2b = jnp.where(keep_first_b, qm, 0)

        r2a = exch(2, red1a.at[pl.ds(s2a, qm), :], st2a, pa2)
        r2b = exch(3, red1b.at[pl.ds(s2b, qm), :], st2b, pa1)
        r2a.wait_recv()
        r2b.wait_recv()

        qoff_a = off_keep_a + k2a
        qoff_b = off_keep_b + k2b
        out_ref[pl.ds(qoff_a, qm), pl.ds(0, hn)] = (
            red1a[pl.ds(k2a, qm), :].astype(jnp.float32)
            + st2a[:, :].astype(jnp.float32)
        ).astype(jnp.bfloat16)
        out_ref[pl.ds(qoff_b, qm), pl.ds(hn, hn)] = (
            red1b[pl.ds(k2b, qm), :].astype(jnp.float32)
            + st2b[:, :].astype(jnp.float32)
        ).astype(jnp.bfloat16)

        r3a = exch(
            4,
            out_ref.at[pl.ds(qoff_a, qm), pl.ds(0, hn)],
            out_ref.at[pl.ds(qoff_a, qm), pl.ds(0, hn)],
            pa2,
        )
        r3b = exch(
            5,
            out_ref.at[pl.ds(qoff_b, qm), pl.ds(hn, hn)],
            out_ref.at[pl.ds(qoff_b, qm), pl.ds(hn, hn)],
            pa1,
        )
        r3a.wait_recv()
        r3b.wait_recv()

        r4a = exch(
            6,
            out_ref.at[pl.ds(off_keep_a, hm), pl.ds(0, hn)],
            out_ref.at[pl.ds(off_keep_a, hm), pl.ds(0, hn)],
            pa1,
        )
        r4b = exch(
            7,
            out_ref.at[pl.ds(off_keep_b, hm), pl.ds(hn, hn)],
            out_ref.at[pl.ds(off_keep_b, hm), pl.ds(hn, hn)],
            pa2,
        )
        r4a.wait_recv()
        r4b.wait_recv()

        for r in sends:
            r.wait_send()

    return pl.pallas_call(
        body,
        out_shape=jax.ShapeDtypeStruct((m, n), jnp.bfloat16),
        in_specs=[pl.BlockSpec(memory_space=pltpu.VMEM)],
        out_specs=pl.BlockSpec(memory_space=pltpu.VMEM),
        scratch_shapes=[
            pltpu.VMEM((m, n), jnp.bfloat16),
            pltpu.VMEM((hm, hn), jnp.bfloat16),
            pltpu.VMEM((hm, hn), jnp.bfloat16),
            pltpu.VMEM((qm, hn), jnp.bfloat16),
            pltpu.VMEM((qm, hn), jnp.bfloat16),
            pltpu.VMEM((hm, hn), jnp.bfloat16),
            pltpu.VMEM((hm, hn), jnp.bfloat16),
            pltpu.SemaphoreType.DMA((8,)),
            pltpu.SemaphoreType.DMA((8,)),
        ],
        compiler_params=pltpu.CompilerParams(collective_id=0),
    )(x)


# device time: 26334 ns/iter; 3.0395x vs baseline; 1.1441x over previous
import jax
import jax.numpy as jnp
from jax import lax
from jax.experimental import pallas as pl
from jax.experimental.pallas import tpu as pltpu

N_DEV = 4
NCHUNK = 2


def kernel(x):
    m, n = x.shape
    hm, hn = m // 2, n // 2
    qm = m // 4
    cw = hn // NCHUNK

    def body(
        x_ref,
        out_ref,
        xbs_ref,
        st1a,
        st1b,
        st2a,
        st2b,
        red1a,
        red1b,
        send_sems,
        recv_sems,
    ):
        my = lax.axis_index("i")
        b0 = my & 1
        b1 = my >> 1
        pa1 = my ^ 1
        pa2 = my ^ 3

        barrier_sem = pltpu.get_barrier_semaphore()
        for nbr in [pa1, pa2]:
            pl.semaphore_signal(
                barrier_sem, inc=1,
                device_id=(nbr,), device_id_type=pl.DeviceIdType.MESH,
            )
        pl.semaphore_wait(barrier_sem, 2)

        keep_lo_a = b0 == b1
        off_keep_a = jnp.where(keep_lo_a, 0, hm)
        off_send_a = jnp.where(keep_lo_a, hm, 0)
        keep_lo_b = b1 == 0
        off_keep_b = jnp.where(keep_lo_b, 0, hm)
        off_send_b = jnp.where(keep_lo_b, hm, 0)
        keep_first_a = b1 == 0
        k2a = jnp.where(keep_first_a, 0, qm)
        s2a = jnp.where(keep_first_a, qm, 0)
        keep_first_b = b0 == 0
        k2b = jnp.where(keep_first_b, 0, qm)
        s2b = jnp.where(keep_first_b, qm, 0)
        qoff_a = off_keep_a + k2a
        qoff_b = off_keep_b + k2b

        xbs_ref[:, 0:hn] = x_ref[pl.ds(off_send_a, hm), 0:hn].astype(jnp.bfloat16)
        xbs_ref[:, hn:n] = x_ref[pl.ds(off_send_b, hm), hn:n].astype(jnp.bfloat16)

        sends = []

        def exch(idx, src, dst, tgt):
            r = pltpu.make_async_remote_copy(
                src_ref=src,
                dst_ref=dst,
                send_sem=send_sems.at[idx],
                recv_sem=recv_sems.at[idx],
                device_id=(tgt,),
                device_id_type=pl.DeviceIdType.MESH,
            )
            r.start()
            sends.append(r)
            return r

        def acols(c):
            return pl.ds(c * cw, cw)

        def bcols(c):
            return pl.ds(hn + c * cw, cw)

        def sem(c, e):
            return c * 8 + e

        r1 = []
        for c in range(NCHUNK):
            ra = exch(sem(c, 0), xbs_ref.at[:, acols(c)], st1a.at[:, acols(c)], pa1)
            rb = exch(sem(c, 1), xbs_ref.at[:, bcols(c)], st1b.at[:, acols(c)], pa2)
            r1.append((ra, rb))

        r2 = [None] * NCHUNK
        for c in range(NCHUNK):
            ra, rb = r1[c]
            ra.wait_recv()
            rb.wait_recv()
            red1a[:, acols(c)] = (
                x_ref[pl.ds(off_keep_a, hm), acols(c)]
                + st1a[:, acols(c)].astype(jnp.float32)
            ).astype(jnp.bfloat16)
            red1b[:, acols(c)] = (
                x_ref[pl.ds(off_keep_b, hm), bcols(c)]
                + st1b[:, acols(c)].astype(jnp.float32)
            ).astype(jnp.bfloat16)
            r2[c] = (
                exch(sem(c, 2), red1a.at[pl.ds(s2a, qm), acols(c)],
                     st2a.at[:, acols(c)], pa2),
                exch(sem(c, 3), red1b.at[pl.ds(s2b, qm), acols(c)],
                     st2b.at[:, acols(c)], pa1),
            )

        r3 = [None] * NCHUNK
        for c in range(NCHUNK):
            ra, rb = r2[c]
            ra.wait_recv()
            rb.wait_recv()
            out_ref[pl.ds(qoff_a, qm), acols(c)] = (
                red1a[pl.ds(k2a, qm), acols(c)].astype(jnp.float32)
                + st2a[:, acols(c)].astype(jnp.float32)
            ).astype(jnp.bfloat16)
            out_ref[pl.ds(qoff_b, qm), bcols(c)] = (
                red1b[pl.ds(k2b, qm), acols(c)].astype(jnp.float32)
                + st2b[:, acols(c)].astype(jnp.float32)
            ).astype(jnp.bfloat16)
            r3[c] = (
                exch(sem(c, 4), out_ref.at[pl.ds(qoff_a, qm), acols(c)],
                     out_ref.at[pl.ds(qoff_a, qm), acols(c)], pa2),
                exch(sem(c, 5), out_ref.at[pl.ds(qoff_b, qm), bcols(c)],
                     out_ref.at[pl.ds(qoff_b, qm), bcols(c)], pa1),
            )

        r4 = [None] * NCHUNK
        for c in range(NCHUNK):
            ra, rb = r3[c]
            ra.wait_recv()
            rb.wait_recv()
            r4[c] = (
                exch(sem(c, 6), out_ref.at[pl.ds(off_keep_a, hm), acols(c)],
                     out_ref.at[pl.ds(off_keep_a, hm), acols(c)], pa1),
                exch(sem(c, 7), out_ref.at[pl.ds(off_keep_b, hm), bcols(c)],
                     out_ref.at[pl.ds(off_keep_b, hm), bcols(c)], pa2),
            )

        for c in range(NCHUNK):
            ra, rb = r4[c]
            ra.wait_recv()
            rb.wait_recv()

        for r in sends:
            r.wait_send()

    return pl.pallas_call(
        body,
        out_shape=jax.ShapeDtypeStruct((m, n), jnp.bfloat16),
        in_specs=[pl.BlockSpec(memory_space=pltpu.VMEM)],
        out_specs=pl.BlockSpec(memory_space=pltpu.VMEM),
        scratch_shapes=[
            pltpu.VMEM((hm, n), jnp.bfloat16),
            pltpu.VMEM((hm, hn), jnp.bfloat16),
            pltpu.VMEM((hm, hn), jnp.bfloat16),
            pltpu.VMEM((qm, hn), jnp.bfloat16),
            pltpu.VMEM((qm, hn), jnp.bfloat16),
            pltpu.VMEM((hm, hn), jnp.bfloat16),
            pltpu.VMEM((hm, hn), jnp.bfloat16),
            pltpu.SemaphoreType.DMA((NCHUNK * 8,)),
            pltpu.SemaphoreType.DMA((NCHUNK * 8,)),
        ],
        compiler_params=pltpu.CompilerParams(collective_id=0),
    )(x)


# device time: 25803 ns/iter; 3.1020x vs baseline; 1.0206x over previous
import jax
import jax.numpy as jnp
from jax import lax
from jax.experimental import pallas as pl
from jax.experimental.pallas import tpu as pltpu

N_DEV = 4
NROW = 2
NCOL = 2


def kernel(x):
    m, n = x.shape
    hn = n // 2
    cw = hn // NCOL
    gm = m // NROW
    hm2 = gm // 2
    qm2 = gm // 4

    def body(
        x_ref,
        out_ref,
        xbs_ref,
        st1a,
        st1b,
        st2a,
        st2b,
        red1a,
        red1b,
        send_sems,
        recv_sems,
    ):
        my = lax.axis_index("i")
        b0 = my & 1
        b1 = my >> 1
        pa1 = my ^ 1
        pa2 = my ^ 3

        barrier_sem = pltpu.get_barrier_semaphore()
        for nbr in [pa1, pa2]:
            pl.semaphore_signal(
                barrier_sem, inc=1,
                device_id=(nbr,), device_id_type=pl.DeviceIdType.MESH,
            )
        pl.semaphore_wait(barrier_sem, 2)

        keep_lo_a = b0 == b1
        off_keep_a = jnp.where(keep_lo_a, 0, hm2)
        off_send_a = jnp.where(keep_lo_a, hm2, 0)
        keep_lo_b = b1 == 0
        off_keep_b = jnp.where(keep_lo_b, 0, hm2)
        off_send_b = jnp.where(keep_lo_b, hm2, 0)
        keep_first_a = b1 == 0
        k2a = jnp.where(keep_first_a, 0, qm2)
        s2a = jnp.where(keep_first_a, qm2, 0)
        keep_first_b = b0 == 0
        k2b = jnp.where(keep_first_b, 0, qm2)
        s2b = jnp.where(keep_first_b, qm2, 0)
        qoff_a = off_keep_a + k2a
        qoff_b = off_keep_b + k2b

        sends = []

        def exch(idx, src, dst, tgt):
            r = pltpu.make_async_remote_copy(
                src_ref=src,
                dst_ref=dst,
                send_sem=send_sems.at[idx],
                recv_sem=recv_sems.at[idx],
                device_id=(tgt,),
                device_id_type=pl.DeviceIdType.MESH,
            )
            r.start()
            sends.append(r)
            return r

        units = [(rc, cc) for rc in range(NROW) for cc in range(NCOL)]

        def ac(cc):
            return pl.ds(cc * cw, cw)

        def bc(cc):
            return pl.ds(hn + cc * cw, cw)

        def sem(u, e):
            return u * 8 + e

        r1 = []
        for u, (rc, cc) in enumerate(units):
            rb = rc * gm
            srow = pl.ds(rc * hm2, hm2)
            xbs_ref[srow, ac(cc)] = x_ref[
                pl.ds(rb + off_send_a, hm2), ac(cc)
            ].astype(jnp.bfloat16)
            xbs_ref[srow, bc(cc)] = x_ref[
                pl.ds(rb + off_send_b, hm2), bc(cc)
            ].astype(jnp.bfloat16)
            r1.append((
                exch(sem(u, 0), xbs_ref.at[srow, ac(cc)],
                     st1a.at[srow, ac(cc)], pa1),
                exch(sem(u, 1), xbs_ref.at[srow, bc(cc)],
                     st1b.at[srow, ac(cc)], pa2),
            ))

        r2 = [None] * len(units)
        for u, (rc, cc) in enumerate(units):
            rb = rc * gm
            srow = pl.ds(rc * hm2, hm2)
            ra, rbx = r1[u]
            ra.wait_recv()
            rbx.wait_recv()
            red1a[srow, ac(cc)] = (
                x_ref[pl.ds(rb + off_keep_a, hm2), ac(cc)]
                + st1a[srow, ac(cc)].astype(jnp.float32)
            ).astype(jnp.bfloat16)
            red1b[srow, ac(cc)] = (
                x_ref[pl.ds(rb + off_keep_b, hm2), bc(cc)]
                + st1b[srow, ac(cc)].astype(jnp.float32)
            ).astype(jnp.bfloat16)
            qrow = pl.ds(rc * qm2, qm2)
            r2[u] = (
                exch(sem(u, 2), red1a.at[pl.ds(rc * hm2 + s2a, qm2), ac(cc)],
                     st2a.at[qrow, ac(cc)], pa2),
                exch(sem(u, 3), red1b.at[pl.ds(rc * hm2 + s2b, qm2), ac(cc)],
                     st2b.at[qrow, ac(cc)], pa1),
            )

        r3 = [None] * len(units)
        for u, (rc, cc) in enumerate(units):
            rb = rc * gm
            qrow = pl.ds(rc * qm2, qm2)
            ra, rbx = r2[u]
            ra.wait_recv()
            rbx.wait_recv()
            out_ref[pl.ds(rb + qoff_a, qm2), ac(cc)] = (
                red1a[pl.ds(rc * hm2 + k2a, qm2), ac(cc)].astype(jnp.float32)
                + st2a[qrow, ac(cc)].astype(jnp.float32)
            ).astype(jnp.bfloat16)
            out_ref[pl.ds(rb + qoff_b, qm2), bc(cc)] = (
                red1b[pl.ds(rc * hm2 + k2b, qm2), ac(cc)].astype(jnp.float32)
                + st2b[qrow, ac(cc)].astype(jnp.float32)
            ).astype(jnp.bfloat16)
            r3[u] = (
                exch(sem(u, 4), out_ref.at[pl.ds(rb + qoff_a, qm2), ac(cc)],
                     out_ref.at[pl.ds(rb + qoff_a, qm2), ac(cc)], pa2),
                exch(sem(u, 5), out_ref.at[pl.ds(rb + qoff_b, qm2), bc(cc)],
                     out_ref.at[pl.ds(rb + qoff_b, qm2), bc(cc)], pa1),
            )

        r4 = [None] * len(units)
        for u, (rc, cc) in enumerate(units):
            rb = rc * gm
            ra, rbx = r3[u]
            ra.wait_recv()
            rbx.wait_recv()
            r4[u] = (
                exch(sem(u, 6), out_ref.at[pl.ds(rb + off_keep_a, hm2), ac(cc)],
                     out_ref.at[pl.ds(rb + off_keep_a, hm2), ac(cc)], pa1),
                exch(sem(u, 7), out_ref.at[pl.ds(rb + off_keep_b, hm2), bc(cc)],
                     out_ref.at[pl.ds(rb + off_keep_b, hm2), bc(cc)], pa2),
            )

        for u in range(len(units)):
            ra, rbx = r4[u]
            ra.wait_recv()
            rbx.wait_recv()

        for r in sends:
            r.wait_send()

    nu = NROW * NCOL
    return pl.pallas_call(
        body,
        out_shape=jax.ShapeDtypeStruct((m, n), jnp.bfloat16),
        in_specs=[pl.BlockSpec(memory_space=pltpu.VMEM)],
        out_specs=pl.BlockSpec(memory_space=pltpu.VMEM),
        scratch_shapes=[
            pltpu.VMEM((NROW * hm2, n), jnp.bfloat16),
            pltpu.VMEM((NROW * hm2, hn), jnp.bfloat16),
            pltpu.VMEM((NROW * hm2, hn), jnp.bfloat16),
            pltpu.VMEM((NROW * qm2, hn), jnp.bfloat16),
            pltpu.VMEM((NROW * qm2, hn), jnp.bfloat16),
            pltpu.VMEM((NROW * hm2, hn), jnp.bfloat16),
            pltpu.VMEM((NROW * hm2, hn), jnp.bfloat16),
            pltpu.SemaphoreType.DMA((nu * 8,)),
            pltpu.SemaphoreType.DMA((nu * 8,)),
        ],
        compiler_params=pltpu.CompilerParams(collective_id=0),
    )(x)
